# baseline (speedup 1.0000x reference)
import functools

import jax
import jax.numpy as jnp
from jax import lax
from jax.experimental import pallas as pl
from jax.experimental.pallas import tpu as pltpu
from jax.experimental.pallas import tpu_sc as plsc

_REP = 4
_ROWS = 2048
_D = 2048
_S = 1024
_NC = 2
_NS = 16
_NW = _NC * _NS
_RPW = _S // _NW
_C = 16
_NCH = _RPW // _C
_BI = 64

_mesh = plsc.VectorSubcoreMesh(core_axis_name="c", subcore_axis_name="s")


@functools.partial(
    pl.kernel,
    mesh=_mesh,
    out_type=jax.ShapeDtypeStruct((_S * _REP, _D), jnp.float32),
    scratch_types=[
        pltpu.VMEM((_C, _D), jnp.float32),
        pltpu.VMEM((_C, _D), jnp.float32),
        pltpu.VMEM((_NCH * _REP, _C), jnp.int32),
        pltpu.SemaphoreType.DMA,
        pltpu.SemaphoreType.DMA,
        pltpu.SemaphoreType.DMA,
        pltpu.SemaphoreType.DMA,
    ],
)
def _upsample_sc(x_hbm, out_hbm, buf0, buf1, idx, rsem0, rsem1, wsem0, wsem1):
    wid = lax.axis_index("s") * _NC + lax.axis_index("c")
    base = wid * _RPW
    bufs = (buf0, buf1)
    rsems = (rsem0, rsem1)
    wsems = (wsem0, wsem1)

    lanes = _REP * lax.iota(jnp.int32, _C)
    for i in range(_NCH):
        for r in range(_REP):
            idx[i * _REP + r] = _REP * base + _REP * _C * i + r + lanes

    def rd(i):
        return pltpu.async_copy(
            x_hbm.at[pl.ds(base + i * _C, _C)], bufs[i % 2], rsems[i % 2])

    reads = {0: rd(0), 1: rd(1)}
    writes = {}
    for i in range(_NCH):
        bi = i % 2
        reads[i].wait()
        writes[i] = [
            pltpu.async_copy(
                bufs[bi], out_hbm.at[idx.at[i * _REP + r]], wsems[bi])
            for r in range(_REP)
        ]
        if i + 2 < _NCH:
            for w in writes[i]:
                w.wait()
            reads[i + 2] = rd(i + 2)
    for w in writes[_NCH - 2] + writes[_NCH - 1]:
        w.wait()


def _tc_body(x_ref, o_ref):
    o_ref[...] = jnp.repeat(x_ref[...], _REP, axis=0)


_upsample_tc = pl.pallas_call(
    _tc_body,
    grid=((_ROWS - _S) // _BI,),
    in_specs=[
        pl.BlockSpec((_BI, _D), lambda i: (_S // _BI + i, 0)),
    ],
    out_specs=pl.BlockSpec((_BI * _REP, _D), lambda i: (i, 0)),
    out_shape=jax.ShapeDtypeStruct(((_ROWS - _S) * _REP, _D), jnp.float32),
)


def kernel(x, x_short):
    lo = _upsample_sc(x_short)
    hi = _upsample_tc(x_short)
    return jnp.concatenate([lo, hi], axis=0)


# speedup vs baseline: 1.8102x; 1.8102x over previous
import functools

import jax
import jax.numpy as jnp
from jax import lax
from jax.experimental import pallas as pl
from jax.experimental.pallas import tpu as pltpu
from jax.experimental.pallas import tpu_sc as plsc

_REP = 4
_ROWS = 2048
_D = 2048
_NC = 2
_NS = 16
_NW = _NC * _NS
_RPW = _ROWS // _NW
_CA = 32
_CB = 16

_mesh = plsc.VectorSubcoreMesh(core_axis_name="c", subcore_axis_name="s")


@functools.partial(
    pl.kernel,
    mesh=_mesh,
    out_type=jax.ShapeDtypeStruct((_ROWS * _REP, _D), jnp.float32),
    scratch_types=[
        pltpu.VMEM((_CA, _D), jnp.float32),
        pltpu.VMEM((_CB, _D), jnp.float32),
        pltpu.VMEM((_REP, _CA), jnp.int32),
        pltpu.VMEM((_REP, _CB), jnp.int32),
        pltpu.VMEM((_REP, _CB), jnp.int32),
        pltpu.SemaphoreType.DMA,
        pltpu.SemaphoreType.DMA,
        pltpu.SemaphoreType.DMA,
        pltpu.SemaphoreType.DMA,
    ],
)
def _upsample(x_hbm, out_hbm, bufa, bufb, idxa, idxb, idxc,
              rsa, rsb, wsa, wsb):
    wid = lax.axis_index("s") * _NC + lax.axis_index("c")
    base = wid * _RPW

    lanes = _REP * lax.iota(jnp.int32, 16)
    for r in range(_REP):
        for k in range(_CA // 16):
            idxa[r, pl.ds(16 * k, 16)] = _REP * (base + 16 * k) + r + lanes
        idxb[r] = _REP * (base + _CA) + r + lanes
        idxc[r] = _REP * (base + _CA + _CB) + r + lanes

    ra = pltpu.async_copy(x_hbm.at[pl.ds(base, _CA)], bufa, rsa)
    rb = pltpu.async_copy(x_hbm.at[pl.ds(base + _CA, _CB)], bufb, rsb)
    ra.wait()
    wa = [pltpu.async_copy(bufa, out_hbm.at[idxa.at[r]], wsa)
          for r in range(_REP)]
    rb.wait()
    wb = [pltpu.async_copy(bufb, out_hbm.at[idxb.at[r]], wsb)
          for r in range(_REP)]
    for w in wa:
        w.wait()
    sub = bufa.at[pl.ds(0, _CB)]
    pltpu.async_copy(x_hbm.at[pl.ds(base + _CA + _CB, _CB)], sub, rsa).wait()
    wc = [pltpu.async_copy(sub, out_hbm.at[idxc.at[r]], wsa)
          for r in range(_REP)]
    for w in wb + wc:
        w.wait()


def kernel(x, x_short):
    return _upsample(x_short)


# speedup vs baseline: 1.8206x; 1.0058x over previous
import functools

import jax
import jax.numpy as jnp
from jax import lax
from jax.experimental import pallas as pl
from jax.experimental.pallas import tpu as pltpu
from jax.experimental.pallas import tpu_sc as plsc

_REP = 4
_ROWS = 2048
_D = 2048
_NC = 2
_NS = 16
_NW = _NC * _NS
_RPW = _ROWS // _NW
_C = 16
_NCH = _RPW // _C

_mesh = plsc.VectorSubcoreMesh(core_axis_name="c", subcore_axis_name="s")


@functools.partial(
    pl.kernel,
    mesh=_mesh,
    out_type=jax.ShapeDtypeStruct((_ROWS * _REP, _D), jnp.float32),
    scratch_types=[
        pltpu.VMEM((_C, _D), jnp.float32),
        pltpu.VMEM((_C, _D), jnp.float32),
        pltpu.VMEM((_NCH * _REP, _C), jnp.int32),
        pltpu.SemaphoreType.DMA,
        pltpu.SemaphoreType.DMA,
        pltpu.SemaphoreType.DMA,
        pltpu.SemaphoreType.DMA,
    ],
)
def _upsample(x_hbm, out_hbm, buf0, buf1, idx, rsem0, rsem1, wsem0, wsem1):
    wid = lax.axis_index("s") * _NC + lax.axis_index("c")
    base = wid * _RPW
    bufs = (buf0, buf1)
    rsems = (rsem0, rsem1)
    wsems = (wsem0, wsem1)

    lanes = _REP * lax.iota(jnp.int32, _C)
    for i in range(_NCH):
        for r in range(_REP):
            idx[i * _REP + r] = _REP * base + _REP * _C * i + r + lanes

    def rd(i):
        return pltpu.async_copy(
            x_hbm.at[pl.ds(base + i * _C, _C)], bufs[i % 2], rsems[i % 2])

    reads = {0: rd(0), 1: rd(1)}
    writes = {}
    for i in range(_NCH):
        bi = i % 2
        reads[i].wait()
        writes[i] = [
            pltpu.async_copy(
                bufs[bi], out_hbm.at[idx.at[i * _REP + r]], wsems[bi])
            for r in range(_REP)
        ]
        if i + 2 < _NCH:
            for w in writes[i]:
                w.wait()
            reads[i + 2] = rd(i + 2)
    for w in writes[_NCH - 2] + writes[_NCH - 1]:
        w.wait()


def kernel(x, x_short):
    return _upsample(x_short)
